# baseline (speedup 1.0000x reference)
import jax
import jax.numpy as jnp
from jax.experimental import pallas as pl
from jax.experimental.pallas import tpu as pltpu

_R = 16
_E = 64
_B = 4
_DIN = 1024
_DOUT = 1024
_SEQ = 2048
_SBLK = 512
_NS = _SEQ // _SBLK
_APPLY0 = 17
_GRID = _APPLY0 + _B * _NS
_OC = 256
_NEG_INF = float("-inf")


def _topk_onehot_idx(scores):
    col = jax.lax.broadcasted_iota(jnp.int32, (_B, _E), 1)
    col128 = jax.lax.broadcasted_iota(jnp.int32, (_B, 128), 1)
    run = scores
    ohs = []
    idx128 = jnp.zeros((_B, 128), dtype=jnp.int32)
    for j in range(_R):
        m = jnp.max(run, axis=1, keepdims=True)
        cand = jnp.where(run == m, col, _E)
        amin = jnp.min(cand, axis=1, keepdims=True)
        oh = col == amin
        ohs.append(oh.astype(jnp.float32))
        run = jnp.where(oh, _NEG_INF, run)
        idx128 = jnp.where(col128 == j, amin, idx128)
    return ohs, idx128


def _bsel_chunk(b_vmem, idxb_s, bs_s, b, o0):
    lane = jax.lax.broadcasted_iota(jnp.int32, (_OC, _R), 1)
    acc = jnp.zeros((_OC, _R), dtype=jnp.float32)
    for r in range(_R):
        e = idxb_s[b, r]
        chunk = b_vmem[e, o0:o0 + _OC, :]
        acc = jnp.where(lane == r, chunk, acc)
    bs_s[b, o0:o0 + _OC, :] = acc


def _fused_kernel(q_ref, wa_ref, ba_ref, wb_ref, bb_ref,
                  a_ref, cfs_ref, x_ref, b_hbm, out_ref,
                  la_s, bs_s, sb_s, oha_s, idxb_v, idxb_s,
                  b_vmem, b_sem, idx_sem):
    i = pl.program_id(0)

    @pl.when(i == 0)
    def _step0():
        pltpu.make_async_copy(b_hbm, b_vmem, b_sem).start()
        q = q_ref[...]
        s_a = jax.lax.dot_general(q, wa_ref[...], (((1,), (1,)), ((), ())),
                                  preferred_element_type=jnp.float32)
        s_a = s_a + ba_ref[...]
        ohs, _ = _topk_onehot_idx(s_a)
        for r in range(_R):
            oha_s[r] = ohs[r]
        s_ifs = jax.lax.dot_general(q, wb_ref[...], (((1,), (1,)), ((), ())),
                                    preferred_element_type=jnp.float32)
        sb_s[...] = s_ifs + bb_ref[...]

    @pl.when(i < _R)
    def _route_step():
        idx = jnp.minimum(i, _R - 1)
        oh_r = oha_s[idx]
        la_r = jnp.dot(oh_r, a_ref[:, idx, :],
                       preferred_element_type=jnp.float32)
        la_s[:, idx, :] = la_r
        sb_s[...] += jnp.dot(la_r, cfs_ref[0],
                             preferred_element_type=jnp.float32)

    @pl.when(i == _R - 1)
    def _finish_scores():
        _, idx128 = _topk_onehot_idx(sb_s[...])
        idxb_v[...] = idx128
        pltpu.make_async_copy(idxb_v, idxb_s, idx_sem).start()

    @pl.when(i == _R)
    def _bsel_b0():
        pltpu.make_async_copy(idxb_v, idxb_s, idx_sem).wait()
        pltpu.make_async_copy(b_hbm, b_vmem, b_sem).wait()
        for c in range(_DOUT // _OC):
            _bsel_chunk(b_vmem, idxb_s, bs_s, 0, c * _OC)

    @pl.when(i >= _APPLY0)
    def _apply():
        j = i - _APPLY0
        b = j // _NS
        x = x_ref[0]
        la = la_s[b]
        bs = bs_s[b]
        after = jax.lax.dot_general(x, la, (((1,), (1,)), ((), ())),
                                    preferred_element_type=jnp.float32)
        out_ref[0] = jax.lax.dot_general(after, bs, (((1,), (1,)), ((), ())),
                                         preferred_element_type=jnp.float32)
        for nb in range(1, _B):
            for c in range(_DOUT // _OC):
                @pl.when(j == (nb - 1) * _NS + c)
                def _spread(nb=nb, c=c):
                    _bsel_chunk(b_vmem, idxb_s, bs_s, nb, c * _OC)


def _run(x, query_signal, A_pool, B_pool, W_A, b_A, W_B, b_B, cfs_W,
         interpret=False):
    n_exp = A_pool.shape[0]
    grid = (_GRID,)
    out = pl.pallas_call(
        _fused_kernel,
        grid=grid,
        in_specs=[
            pl.BlockSpec((_B, _DIN), lambda i: (0, 0)),
            pl.BlockSpec((_E, _DIN), lambda i: (0, 0)),
            pl.BlockSpec((1, _E), lambda i: (0, 0)),
            pl.BlockSpec((_E, _DIN), lambda i: (0, 0)),
            pl.BlockSpec((1, _E), lambda i: (0, 0)),
            pl.BlockSpec((_E, _R, _DIN), lambda i: (0, 0, 0)),
            pl.BlockSpec((1, _DIN, _E),
                         lambda i: (jnp.minimum(i, _R - 1), 0, 0)),
            pl.BlockSpec((1, _SBLK, _DIN),
                         lambda i: (jnp.maximum(i - _APPLY0, 0) // _NS,
                                    jnp.maximum(i - _APPLY0, 0) % _NS, 0)),
            pl.BlockSpec(memory_space=pl.ANY),
        ],
        out_specs=pl.BlockSpec(
            (1, _SBLK, _DOUT),
            lambda i: (jnp.maximum(i - _APPLY0, 0) // _NS,
                       jnp.maximum(i - _APPLY0, 0) % _NS, 0)),
        out_shape=jax.ShapeDtypeStruct((_B, _SEQ, _DOUT), jnp.float32),
        scratch_shapes=[
            pltpu.VMEM((_B, _R, _DIN), jnp.float32),
            pltpu.VMEM((_B, _DOUT, _R), jnp.float32),
            pltpu.VMEM((_B, _E), jnp.float32),
            pltpu.VMEM((_R, _B, _E), jnp.float32),
            pltpu.VMEM((_B, 128), jnp.int32),
            pltpu.SMEM((_B, 128), jnp.int32),
            pltpu.VMEM((n_exp, _DOUT, _R), jnp.float32),
            pltpu.SemaphoreType.DMA,
            pltpu.SemaphoreType.DMA,
        ],
        compiler_params=pltpu.CompilerParams(
            dimension_semantics=("arbitrary",)),
        interpret=interpret,
    )(query_signal, W_A, b_A.reshape(1, n_exp), W_B, b_B.reshape(1, n_exp),
      A_pool, cfs_W, x, B_pool)
    return out


def kernel(x, query_signal, A_pool, B_pool, W_A, b_A, W_B, b_B, cfs_W):
    return _run(x, query_signal, A_pool, B_pool, W_A, b_A, W_B, b_B, cfs_W)


# speedup vs baseline: 1.3761x; 1.3761x over previous
import jax
import jax.numpy as jnp
from jax.experimental import pallas as pl
from jax.experimental.pallas import tpu as pltpu

_R = 16
_E = 64
_B = 4
_DIN = 1024
_DOUT = 1024
_SEQ = 2048
_APPLY0 = 17
_GRID = _APPLY0 + _B
_NEG_INF = float("-inf")


def _topk_onehots(scores):
    col = jax.lax.broadcasted_iota(jnp.int32, (_B, _E), 1)
    run = scores
    ohs = []
    for _ in range(_R):
        m = jnp.max(run, axis=1, keepdims=True)
        cand = jnp.where(run == m, col, _E)
        amin = jnp.min(cand, axis=1, keepdims=True)
        oh = col == amin
        ohs.append(oh.astype(jnp.float32))
        run = jnp.where(oh, _NEG_INF, run)
    return ohs


def _fused_kernel(q_ref, wa_ref, ba_ref, wb_ref, bb_ref,
                  a_ref, cfs_ref, bt_ref, x_ref, out_ref,
                  la_s, bs_s, sb_s, oha_s, ohb_s):
    i = pl.program_id(0)

    @pl.when(i == 0)
    def _step0():
        q = q_ref[...]
        s_a = jax.lax.dot_general(q, wa_ref[...], (((1,), (1,)), ((), ())),
                                  preferred_element_type=jnp.float32)
        s_a = s_a + ba_ref[...]
        ohs = _topk_onehots(s_a)
        for r in range(_R):
            oha_s[r] = ohs[r]
        s_ifs = jax.lax.dot_general(q, wb_ref[...], (((1,), (1,)), ((), ())),
                                    preferred_element_type=jnp.float32)
        sb_s[...] = s_ifs + bb_ref[...]

    @pl.when(i < _R)
    def _route_step():
        idx = jnp.minimum(i, _R - 1)
        oh_r = oha_s[idx]
        la_r = jnp.dot(oh_r, a_ref[:, idx, :],
                       preferred_element_type=jnp.float32)
        la_s[:, idx, :] = la_r
        sb_s[...] += jnp.dot(la_r, cfs_ref[0],
                             preferred_element_type=jnp.float32)

    @pl.when(i == _R - 1)
    def _finish_scores():
        ohs = _topk_onehots(sb_s[...])
        for r in range(_R):
            ohb_s[r] = ohs[r]

    @pl.when(i == _R)
    def _bsel():
        for r in range(_R):
            bs_s[:, r, :] = jnp.dot(ohb_s[r], bt_ref[r],
                                    preferred_element_type=jnp.float32)

    @pl.when(i >= _APPLY0)
    def _apply():
        b = i - _APPLY0
        x = x_ref[0]
        la = la_s[b]
        bs = bs_s[b]
        after = jax.lax.dot_general(x, la, (((1,), (1,)), ((), ())),
                                    preferred_element_type=jnp.float32)
        out_ref[0] = jnp.dot(after, bs, preferred_element_type=jnp.float32)


def _run(x, query_signal, A_pool, B_pool, W_A, b_A, W_B, b_B, cfs_W,
         interpret=False):
    n_exp = A_pool.shape[0]
    bt = jnp.transpose(B_pool, (2, 0, 1))
    out = pl.pallas_call(
        _fused_kernel,
        grid=(_GRID,),
        in_specs=[
            pl.BlockSpec((_B, _DIN), lambda i: (0, 0)),
            pl.BlockSpec((_E, _DIN), lambda i: (0, 0)),
            pl.BlockSpec((1, _E), lambda i: (0, 0)),
            pl.BlockSpec((_E, _DIN), lambda i: (0, 0)),
            pl.BlockSpec((1, _E), lambda i: (0, 0)),
            pl.BlockSpec((_E, _R, _DIN), lambda i: (0, 0, 0)),
            pl.BlockSpec((1, _DIN, _E),
                         lambda i: (jnp.minimum(i, _R - 1), 0, 0)),
            pl.BlockSpec((_R, _E, _DOUT), lambda i: (0, 0, 0)),
            pl.BlockSpec((1, _SEQ, _DIN),
                         lambda i: (jnp.maximum(i - _APPLY0, 0), 0, 0)),
        ],
        out_specs=pl.BlockSpec(
            (1, _SEQ, _DOUT),
            lambda i: (jnp.maximum(i - _APPLY0, 0), 0, 0)),
        out_shape=jax.ShapeDtypeStruct((_B, _SEQ, _DOUT), jnp.float32),
        scratch_shapes=[
            pltpu.VMEM((_B, _R, _DIN), jnp.float32),
            pltpu.VMEM((_B, _R, _DOUT), jnp.float32),
            pltpu.VMEM((_B, _E), jnp.float32),
            pltpu.VMEM((_R, _B, _E), jnp.float32),
            pltpu.VMEM((_R, _B, _E), jnp.float32),
        ],
        compiler_params=pltpu.CompilerParams(
            dimension_semantics=("arbitrary",)),
        interpret=interpret,
    )(query_signal, W_A, b_A.reshape(1, n_exp), W_B, b_B.reshape(1, n_exp),
      A_pool, cfs_W, bt, x)
    return out


def kernel(x, query_signal, A_pool, B_pool, W_A, b_A, W_B, b_B, cfs_W):
    return _run(x, query_signal, A_pool, B_pool, W_A, b_A, W_B, b_B, cfs_W)


# speedup vs baseline: 1.5005x; 1.0904x over previous
import jax
import jax.numpy as jnp
from jax.experimental import pallas as pl
from jax.experimental.pallas import tpu as pltpu

_R = 16
_E = 64
_B = 4
_DIN = 1024
_DOUT = 1024
_SEQ = 2048
_NEG_INF = float("-inf")


def _topk_onehots(scores):
    col = jax.lax.broadcasted_iota(jnp.int32, (_B, _E), 1)
    run = scores
    ohs = []
    for _ in range(_R):
        m = jnp.max(run, axis=1, keepdims=True)
        cand = jnp.where(run == m, col, _E)
        amin = jnp.min(cand, axis=1, keepdims=True)
        oh = col == amin
        ohs.append(oh.astype(jnp.float32))
        run = jnp.where(oh, _NEG_INF, run)
    return ohs


def _fused_kernel(q_ref, wa_ref, ba_ref, wb_ref, bb_ref,
                  at_ref, cfs_ref, bt_ref, x_ref, out_ref,
                  la_s, bs_s):
    i = pl.program_id(0)

    @pl.when(i == 0)
    def _route():
        q = q_ref[...]
        s_a = jax.lax.dot_general(q, wa_ref[...], (((1,), (1,)), ((), ())),
                                  preferred_element_type=jnp.float32)
        oh_a = _topk_onehots(s_a + ba_ref[...])
        s_b = jax.lax.dot_general(q, wb_ref[...], (((1,), (1,)), ((), ())),
                                  preferred_element_type=jnp.float32)
        s_b = s_b + bb_ref[...]
        for r in range(_R):
            la_r = jnp.dot(oh_a[r], at_ref[r],
                           preferred_element_type=jnp.float32)
            la_s[:, r, :] = la_r
            s_b = s_b + jnp.dot(la_r, cfs_ref[r],
                                preferred_element_type=jnp.float32)
        oh_b = _topk_onehots(s_b)
        for r in range(_R):
            bs_s[:, r, :] = jnp.dot(oh_b[r], bt_ref[r],
                                    preferred_element_type=jnp.float32)

    for b in range(_B):
        @pl.when(i == 1 + b)
        def _apply(b=b):
            x = x_ref[0]
            after = jax.lax.dot_general(
                x, la_s[b], (((1,), (1,)), ((), ())),
                preferred_element_type=jnp.float32)
            out_ref[0] = jnp.dot(after, bs_s[b],
                                 preferred_element_type=jnp.float32)


def _run(x, query_signal, A_pool, B_pool, W_A, b_A, W_B, b_B, cfs_W,
         interpret=False):
    n_exp = A_pool.shape[0]
    at = jnp.transpose(A_pool, (1, 0, 2))
    bt = jnp.transpose(B_pool, (2, 0, 1))
    out = pl.pallas_call(
        _fused_kernel,
        grid=(1 + _B,),
        in_specs=[
            pl.BlockSpec((_B, _DIN), lambda i: (0, 0)),
            pl.BlockSpec((_E, _DIN), lambda i: (0, 0)),
            pl.BlockSpec((1, _E), lambda i: (0, 0)),
            pl.BlockSpec((_E, _DIN), lambda i: (0, 0)),
            pl.BlockSpec((1, _E), lambda i: (0, 0)),
            pl.BlockSpec((_R, _E, _DIN), lambda i: (0, 0, 0)),
            pl.BlockSpec((_R, _DIN, _E), lambda i: (0, 0, 0)),
            pl.BlockSpec((_R, _E, _DOUT), lambda i: (0, 0, 0)),
            pl.BlockSpec((1, _SEQ, _DIN),
                         lambda i: (jnp.maximum(i - 1, 0), 0, 0)),
        ],
        out_specs=pl.BlockSpec(
            (1, _SEQ, _DOUT),
            lambda i: (jnp.maximum(i - 1, 0), 0, 0)),
        out_shape=jax.ShapeDtypeStruct((_B, _SEQ, _DOUT), jnp.float32),
        scratch_shapes=[
            pltpu.VMEM((_B, _R, _DIN), jnp.float32),
            pltpu.VMEM((_B, _R, _DOUT), jnp.float32),
        ],
        compiler_params=pltpu.CompilerParams(
            dimension_semantics=("arbitrary",)),
        interpret=interpret,
    )(query_signal, W_A, b_A.reshape(1, n_exp), W_B, b_B.reshape(1, n_exp),
      at, cfs_W, bt, x)
    return out


def kernel(x, query_signal, A_pool, B_pool, W_A, b_A, W_B, b_B, cfs_W):
    return _run(x, query_signal, A_pool, B_pool, W_A, b_A, W_B, b_B, cfs_W)


# speedup vs baseline: 1.6702x; 1.1130x over previous
import jax
import jax.numpy as jnp
from jax.experimental import pallas as pl
from jax.experimental.pallas import tpu as pltpu

_R = 16
_E = 64
_B = 4
_DIN = 1024
_DOUT = 1024
_SEQ = 2048
_NEG_INF = float("-inf")


def _topk_onehots(scores):
    col = jax.lax.broadcasted_iota(jnp.int32, (_B, _E), 1)
    run = scores
    ohs = []
    for _ in range(_R):
        m = jnp.max(run, axis=1, keepdims=True)
        cand = jnp.where(run == m, col, _E)
        amin = jnp.min(cand, axis=1, keepdims=True)
        oh = col == amin
        ohs.append(oh.astype(jnp.float32))
        run = jnp.where(oh, _NEG_INF, run)
    return ohs


def _fused_kernel(q_ref, wa_ref, ba_ref, wb_ref, bb_ref,
                  at_ref, cfs_ref, bt_ref, x_ref, out_ref,
                  la_s, bs_s):
    i = pl.program_id(0)

    @pl.when(i == 0)
    def _route():
        q = q_ref[...]
        s_a = jax.lax.dot_general(q, wa_ref[...], (((1,), (1,)), ((), ())),
                                  preferred_element_type=jnp.float32)
        oh_a = _topk_onehots(s_a + ba_ref[...])
        s_b = jax.lax.dot_general(q, wb_ref[...], (((1,), (1,)), ((), ())),
                                  preferred_element_type=jnp.float32)
        s_b = s_b + bb_ref[...]
        for r in range(_R):
            la_r = jnp.dot(oh_a[r], at_ref[r],
                           preferred_element_type=jnp.float32)
            la_s[:, r, :] = la_r
            s_b = s_b + jax.lax.dot_general(
                la_r, cfs_ref[r], (((1,), (1,)), ((), ())),
                preferred_element_type=jnp.float32)
        oh_b = _topk_onehots(s_b)
        for r in range(_R):
            bs_s[:, r, :] = jnp.dot(oh_b[r], bt_ref[r],
                                    preferred_element_type=jnp.float32)

    for b in range(_B):
        @pl.when(i == 1 + b)
        def _apply(b=b):
            x = x_ref[0]
            after = jax.lax.dot_general(
                x, la_s[b], (((1,), (1,)), ((), ())),
                preferred_element_type=jnp.float32)
            out_ref[0] = jnp.dot(after, bs_s[b],
                                 preferred_element_type=jnp.float32)


def _run(x, query_signal, A_pool, B_pool, W_A, b_A, W_B, b_B, cfs_W,
         interpret=False):
    n_exp = A_pool.shape[0]
    at = jnp.transpose(A_pool, (1, 0, 2))
    bt = jnp.transpose(B_pool, (2, 0, 1))
    cfst = jnp.transpose(cfs_W, (0, 2, 1))
    out = pl.pallas_call(
        _fused_kernel,
        grid=(1 + _B,),
        in_specs=[
            pl.BlockSpec((_B, _DIN), lambda i: (0, 0)),
            pl.BlockSpec((_E, _DIN), lambda i: (0, 0)),
            pl.BlockSpec((1, _E), lambda i: (0, 0)),
            pl.BlockSpec((_E, _DIN), lambda i: (0, 0)),
            pl.BlockSpec((1, _E), lambda i: (0, 0)),
            pl.BlockSpec((_R, _E, _DIN), lambda i: (0, 0, 0)),
            pl.BlockSpec((_R, _E, _DIN), lambda i: (0, 0, 0)),
            pl.BlockSpec((_R, _E, _DOUT), lambda i: (0, 0, 0)),
            pl.BlockSpec((1, _SEQ, _DIN),
                         lambda i: (jnp.maximum(i - 1, 0), 0, 0)),
        ],
        out_specs=pl.BlockSpec(
            (1, _SEQ, _DOUT),
            lambda i: (jnp.maximum(i - 1, 0), 0, 0)),
        out_shape=jax.ShapeDtypeStruct((_B, _SEQ, _DOUT), jnp.float32),
        scratch_shapes=[
            pltpu.VMEM((_B, _R, _DIN), jnp.float32),
            pltpu.VMEM((_B, _R, _DOUT), jnp.float32),
        ],
        compiler_params=pltpu.CompilerParams(
            dimension_semantics=("arbitrary",)),
        interpret=interpret,
    )(query_signal, W_A, b_A.reshape(1, n_exp), W_B, b_B.reshape(1, n_exp),
      at, cfst, bt, x)
    return out


def kernel(x, query_signal, A_pool, B_pool, W_A, b_A, W_B, b_B, cfs_W):
    return _run(x, query_signal, A_pool, B_pool, W_A, b_A, W_B, b_B, cfs_W)


# speedup vs baseline: 1.8025x; 1.0793x over previous
import jax
import jax.numpy as jnp
from jax.experimental import pallas as pl
from jax.experimental.pallas import tpu as pltpu

_R = 16
_E = 64
_B = 4
_DIN = 1024
_DOUT = 1024
_SEQ = 2048
_SBLK = 1024
_NH = _SEQ // _SBLK
_NEG_INF = float("-inf")


def _topk_onehots(scores):
    col = jax.lax.broadcasted_iota(jnp.int32, (_B, _E), 1)
    run = scores
    ohs = []
    for _ in range(_R):
        m = jnp.max(run, axis=1, keepdims=True)
        cand = jnp.where(run == m, col, _E)
        amin = jnp.min(cand, axis=1, keepdims=True)
        oh = col == amin
        ohs.append(oh.astype(jnp.float32))
        run = jnp.where(oh, _NEG_INF, run)
    return ohs


def _fused_kernel(q_ref, wa_ref, ba_ref, wb_ref, bb_ref,
                  at_ref, cfs_ref, bt_ref, x_ref, out_ref,
                  la_s, bs_s):
    i = pl.program_id(0)

    @pl.when(i == 0)
    def _route():
        q = q_ref[...]
        s_a = jax.lax.dot_general(q, wa_ref[...], (((1,), (1,)), ((), ())),
                                  preferred_element_type=jnp.float32)
        oh_a = _topk_onehots(s_a + ba_ref[...])
        s_b = jax.lax.dot_general(q, wb_ref[...], (((1,), (1,)), ((), ())),
                                  preferred_element_type=jnp.float32)
        s_b = s_b + bb_ref[...]
        for r in range(_R):
            la_r = jnp.dot(oh_a[r], at_ref[:, r, :],
                           preferred_element_type=jnp.float32)
            la_s[:, r, :] = la_r
            s_b = s_b + jax.lax.dot_general(
                la_r, cfs_ref[r], (((1,), (1,)), ((), ())),
                preferred_element_type=jnp.float32)
        oh_b = _topk_onehots(s_b)
        for r in range(_R):
            bs_s[:, r, :] = jnp.dot(oh_b[r], bt_ref[r],
                                    preferred_element_type=jnp.float32)

    for b in range(_B):
        for h in range(_NH):
            @pl.when(i == 1 + b * _NH + h)
            def _apply(b=b):
                x = x_ref[0]
                after = jax.lax.dot_general(
                    x, la_s[b], (((1,), (1,)), ((), ())),
                    preferred_element_type=jnp.float32)
                out_ref[0] = jnp.dot(after, bs_s[b],
                                     preferred_element_type=jnp.float32)


def _run(x, query_signal, A_pool, B_pool, W_A, b_A, W_B, b_B, cfs_W,
         interpret=False):
    n_exp = A_pool.shape[0]
    bt = jnp.transpose(B_pool, (2, 0, 1))
    cfst = jnp.transpose(cfs_W, (0, 2, 1))
    out = pl.pallas_call(
        _fused_kernel,
        grid=(1 + _B * _NH,),
        in_specs=[
            pl.BlockSpec((_B, _DIN), lambda i: (0, 0)),
            pl.BlockSpec((_E, _DIN), lambda i: (0, 0)),
            pl.BlockSpec((1, _E), lambda i: (0, 0)),
            pl.BlockSpec((_E, _DIN), lambda i: (0, 0)),
            pl.BlockSpec((1, _E), lambda i: (0, 0)),
            pl.BlockSpec((_E, _R, _DIN), lambda i: (0, 0, 0)),
            pl.BlockSpec((_R, _E, _DIN), lambda i: (0, 0, 0)),
            pl.BlockSpec((_R, _E, _DOUT), lambda i: (0, 0, 0)),
            pl.BlockSpec((1, _SBLK, _DIN),
                         lambda i: (jnp.maximum(i - 1, 0) // _NH,
                                    jnp.maximum(i - 1, 0) % _NH, 0)),
        ],
        out_specs=pl.BlockSpec(
            (1, _SBLK, _DOUT),
            lambda i: (jnp.maximum(i - 1, 0) // _NH,
                       jnp.maximum(i - 1, 0) % _NH, 0)),
        out_shape=jax.ShapeDtypeStruct((_B, _SEQ, _DOUT), jnp.float32),
        scratch_shapes=[
            pltpu.VMEM((_B, _R, _DIN), jnp.float32),
            pltpu.VMEM((_B, _R, _DOUT), jnp.float32),
        ],
        compiler_params=pltpu.CompilerParams(
            dimension_semantics=("arbitrary",)),
        interpret=interpret,
    )(query_signal, W_A, b_A.reshape(1, n_exp), W_B, b_B.reshape(1, n_exp),
      A_pool, cfst, bt, x)
    return out


def kernel(x, query_signal, A_pool, B_pool, W_A, b_A, W_B, b_B, cfs_W):
    return _run(x, query_signal, A_pool, B_pool, W_A, b_A, W_B, b_B, cfs_W)
